# Initial kernel scaffold; baseline (speedup 1.0000x reference)
import jax
import jax.numpy as jnp
from jax.experimental import pallas as pl


def kernel(feature):
    raise NotImplementedError("write your pallas kernel here")


# speedup vs baseline: 2.2798x; 2.2798x over previous
import functools
import math
import random

import numpy as np
import jax
import jax.numpy as jnp
from jax import lax
from jax.experimental import pallas as pl
from jax.experimental.pallas import tpu as pltpu
from jax.experimental.pallas import tpu_sc as plsc

LANES = 16


def _distortion_indices(h, w, a_max=3, r_max=0.7):
    random.seed(0)
    cols = h
    rows = w
    center_rows = int(np.round(random.uniform(1, rows - 2)))
    center_cols = int(np.round(random.uniform(1, cols - 2)))
    radius = random.uniform(0.03 * max(rows, cols), r_max * max(rows, cols))
    choice = random.randint(0, 1)
    spect_ratio1 = 1
    spect_ratio2 = 1
    if choice == 1:
        spect_ratio1 = random.uniform(1, a_max)
    else:
        spect_ratio2 = random.uniform(1, a_max)
    cols_np = np.arange(cols)
    rows_np = np.arange(rows)
    cols_np_t = np.tile(cols_np, (rows, 1))
    cols_pow = np.power(cols_np_t - center_cols, 2)
    rows_np_t = np.tile(rows_np, (cols, 1))
    rows_pow = np.power(rows_np_t - center_rows, 2)
    dis = np.sqrt(cols_pow + rows_pow.transpose())
    judge = (spect_ratio1 * np.abs(rows_np_t - center_rows).transpose()
             + spect_ratio2 * np.abs(cols_np_t - center_cols))
    index = np.where(judge <= radius)
    index_rows = np.rint(index[0]).astype(np.int64)
    index_cols = np.rint(index[1]).astype(np.int64)
    dis_val = dis[index]
    old_i = np.floor(dis_val / radius * (index_rows - center_rows) + center_rows)
    old_j = np.floor(dis_val / radius * (index_cols - center_cols) + center_cols)
    return (index_rows, index_cols,
            old_i.astype(np.int64), old_j.astype(np.int64))


def _flat_patch_indices(h, w):
    ir, ic, oi, oj = _distortion_indices(h, w)
    oi = np.where(oi < 0, oi + h, oi).clip(0, h - 1)
    oj = np.where(oj < 0, oj + w, oj).clip(0, w - 1)
    src = (oi * w + oj).astype(np.int32)
    dst = (ir * w + ic).astype(np.int32)
    k = src.shape[0]
    k_pad = math.ceil(k / LANES) * LANES
    src = np.concatenate([src, np.full(k_pad - k, src[-1], np.int32)])
    dst = np.concatenate([dst, np.full(k_pad - k, dst[-1], np.int32)])
    return src, dst


@functools.cache
def _build_sc_call(n_slices, hw, k_pad):
    info = plsc.get_sparse_core_info()
    nc, ns = info.num_cores, info.num_subcores
    n_workers = nc * ns
    assert n_slices % n_workers == 0
    per_worker = n_slices // n_workers
    n_chunks = k_pad // LANES
    mesh = plsc.VectorSubcoreMesh(core_axis_name="c", subcore_axis_name="s")

    @functools.partial(
        pl.kernel,
        mesh=mesh,
        out_type=jax.ShapeDtypeStruct((n_slices, hw), jnp.float32),
        compiler_params=pltpu.CompilerParams(needs_layout_passes=False),
        scratch_types=[
            pltpu.VMEM((k_pad,), jnp.int32),
            pltpu.VMEM((k_pad,), jnp.int32),
            pltpu.VMEM((k_pad,), jnp.float32),
            pltpu.VMEM((hw,), jnp.float32),
        ],
    )
    def sc_patch(feat_hbm, src_hbm, dst_hbm, out_hbm,
                 src_v, dst_v, vals_v, slice_v):
        wid = lax.axis_index("s") * nc + lax.axis_index("c")
        pltpu.sync_copy(src_hbm, src_v)
        pltpu.sync_copy(dst_hbm, dst_v)

        def do_slice(i, _):
            sl = wid * per_worker + i
            pltpu.sync_copy(feat_hbm.at[sl], slice_v)

            def gather_chunk(t, _):
                idx = src_v[pl.ds(t * LANES, LANES)]
                vals_v[pl.ds(t * LANES, LANES)] = plsc.load_gather(
                    slice_v, [idx])
                return 0

            lax.fori_loop(0, n_chunks, gather_chunk, 0, unroll=4)

            def scatter_chunk(t, _):
                idx = dst_v[pl.ds(t * LANES, LANES)]
                plsc.store_scatter(slice_v, [idx],
                                   vals_v[pl.ds(t * LANES, LANES)])
                return 0

            lax.fori_loop(0, n_chunks, scatter_chunk, 0, unroll=4)
            pltpu.sync_copy(slice_v, out_hbm.at[sl])
            return 0

        lax.fori_loop(0, per_worker, do_slice, 0)

    return sc_patch


def kernel(feature):
    b, c, h, w = feature.shape
    src, dst = _flat_patch_indices(h, w)
    n_slices, hw = b * c, h * w
    sc_patch = _build_sc_call(n_slices, hw, src.shape[0])
    out = sc_patch(feature.reshape(n_slices, hw),
                   jnp.asarray(src), jnp.asarray(dst))
    return out.reshape(b, c, h, w)
